# baseline (speedup 1.0000x reference)
import functools

import jax
import jax.numpy as jnp
from jax import lax
from jax.experimental import pallas as pl
from jax.experimental.pallas import tpu as pltpu
from jax.experimental.pallas import tpu_sc as plsc

N = 50000
E = 800000
NP = 51200
EP = 819200
CW = 32
BN = 40
BE = 128
NC, NS = 2, 16
RW = NP // (NC * NS)
RT = NP // NS
EPB = EP // BE
ETR = EP // NS // BE
EHR = EP // (NC * NS) // BE
IB = 40
RB = 2048
GN = NP // RB

_mesh = plsc.VectorSubcoreMesh(core_axis_name="c", subcore_axis_name="s")
_scp = pltpu.CompilerParams(use_tc_tiling_on_sc=False)


@functools.partial(
    pl.kernel,
    out_type=(
        jax.ShapeDtypeStruct((NP, 128), jnp.float32),
        jax.ShapeDtypeStruct((NC * NP,), jnp.float32),
    ),
    mesh=_mesh,
    compiler_params=_scp,
    scratch_types=[
        pltpu.VMEM((RW,), jnp.int32),
        pltpu.VMEM((RW, CW), jnp.float32),
        pltpu.VMEM((EHR, BE), jnp.int32),
        pltpu.VMEM((BE,), jnp.float32),
        pltpu.VMEM_SHARED((NP,), jnp.float32),
        pltpu.SemaphoreType.DMA,
    ],
)
def _embed_deg(cat_h, sub_h, elem_h, dst_h, ones_h, zeros1_h,
               tcat_h, tsub_h, telem_h,
               x_h, degp_h,
               idx_v, rows_v, eidx_v, ones_v, deg_acc, sem):
    cid = lax.axis_index("c")
    tid = lax.axis_index("s")
    wid = tid * NC + cid

    row0 = pl.multiple_of(tid * RT, 8)
    pltpu.sync_copy(zeros1_h, deg_acc.at[pl.ds(row0, RT)])
    pltpu.sync_copy(ones_h, ones_v)
    ebase = pl.multiple_of((cid * NS + tid) * EHR, 8)
    pltpu.sync_copy(dst_h.at[pl.ds(ebase, EHR), :], eidx_v)
    plsc.subcore_barrier()

    def dbody(blk, c):
        descs = []
        for j in range(25):
            descs.append(pltpu.async_copy(
                ones_v, deg_acc.at[eidx_v.at[blk * 25 + j]], sem, add=True))
        for d in descs:
            d.wait()
        return c
    lax.fori_loop(0, EHR // 25, dbody, 0)

    base = pl.multiple_of(wid * RW, 8)
    for t, (tab_h, nidx_h) in enumerate(((tcat_h, cat_h), (tsub_h, sub_h),
                                         (telem_h, elem_h))):
        pltpu.sync_copy(nidx_h.at[pl.ds(base, RW)], idx_v)
        cps = [
            pltpu.async_copy(tab_h.at[idx_v.at[pl.ds(j * BN, BN)]],
                             rows_v.at[pl.ds(j * BN, BN), :], sem)
            for j in range(RW // BN)
        ]
        for cp in cps:
            cp.wait()
        pltpu.sync_copy(rows_v,
                        x_h.at[pl.ds(base, RW), pl.ds(t * CW, CW)])

    plsc.subcore_barrier()
    orow = pl.multiple_of(cid * NP + tid * RT, 8)
    pltpu.sync_copy(deg_acc.at[pl.ds(row0, RT)], degp_h.at[pl.ds(orow, RT)])


def _make_agg(C):
    @functools.partial(
        pl.kernel,
        out_type=(
            jax.ShapeDtypeStruct((NP, 128), jnp.float32),
            jax.ShapeDtypeStruct((C * NP, CW), jnp.float32),
        ),
        mesh=_mesh,
        compiler_params=_scp,
        scratch_types=[
            pltpu.VMEM((IB, BE), jnp.int32),
            pltpu.VMEM((IB, BE), jnp.int32),
            pltpu.VMEM((4, BE), jnp.int32),
            pltpu.VMEM((4, BE, CW), jnp.float32),
            pltpu.VMEM_SHARED((NP, CW), jnp.float32),
            pltpu.SemaphoreType.DMA,
        ],
    )
    def agg(y_h, src_h, dst_h, zeros_h, s_h, ychk_h,
            src_v, dst_v, adj_v, rows_v, acc, sem):
        cid = lax.axis_index("c")
        tid = lax.axis_index("s")
        cpc = C // 2
        row0 = pl.multiple_of(tid * RT, 8)
        tb = pl.multiple_of(tid * ETR, 8)

        for k in range(cpc):
            chunk = cid * cpc + k
            crow = pl.multiple_of(chunk * NP + tid * RT, 8)
            pltpu.sync_copy(
                y_h.at[pl.ds(row0, RT), pl.ds(chunk * CW, CW)],
                ychk_h.at[pl.ds(crow, RT), :])
        plsc.subcore_barrier()

        def adjust(j, slot, chunk):
            for q in range(BE // 16):
                adj_v[slot, pl.ds(q * 16, 16)] = (
                    src_v[j, pl.ds(q * 16, 16)] + chunk * NP)

        for k in range(cpc):
            chunk = cid * cpc + k
            pltpu.sync_copy(zeros_h, acc.at[pl.ds(row0, RT), :])
            plsc.subcore_barrier()

            def blk_body(blk, c):
                boff = blk * IB
                pltpu.sync_copy(src_h.at[pl.ds(tb + boff, IB), :], src_v)
                pltpu.sync_copy(dst_h.at[pl.ds(tb + boff, IB), :], dst_v)
                descs = {}
                for j in range(3):
                    adjust(j, j % 4, chunk)
                    descs[j] = pltpu.async_copy(
                        ychk_h.at[adj_v.at[j % 4]], rows_v.at[j % 4], sem)
                for j in range(IB):
                    descs[j].wait()
                    if j + 3 < IB:
                        adjust(j + 3, (j + 3) % 4, chunk)
                        descs[j + 3] = pltpu.async_copy(
                            ychk_h.at[adj_v.at[(j + 3) % 4]],
                            rows_v.at[(j + 3) % 4], sem)
                    pltpu.sync_copy(rows_v.at[j % 4], acc.at[dst_v.at[j]],
                                    add=True)
                return c
            lax.fori_loop(0, ETR // IB, blk_body, 0)
            plsc.subcore_barrier()

            pltpu.sync_copy(
                acc.at[pl.ds(row0, RT), :],
                s_h.at[pl.ds(row0, RT), pl.ds(chunk * CW, CW)])
    return agg


_agg4 = _make_agg(4)
_agg2 = _make_agg(2)


def _y1_body(x, dg0, dg1, w1, y1o):
    dinv = lax.rsqrt(dg0[...] + dg1[...] + 1.0)
    xv = x[...]
    xw = (jnp.dot(xv[:, 0:CW], w1[0:CW, :],
                  preferred_element_type=jnp.float32)
          + jnp.dot(xv[:, CW:2 * CW], w1[CW:2 * CW, :],
                    preferred_element_type=jnp.float32)
          + jnp.dot(xv[:, 2 * CW:3 * CW], w1[2 * CW:3 * CW, :],
                    preferred_element_type=jnp.float32))
    y1o[...] = xw * dinv[:, None]


def _y2_body(s1, y1, dg0, dg1, b1, w2, y2o):
    dinv = lax.rsqrt(dg0[...] + dg1[...] + 1.0)
    t = jnp.maximum((s1[...] + y1[...]) * dinv[:, None] + b1[...][None, :],
                    0.0)
    y2 = jnp.dot(t, w2[...], preferred_element_type=jnp.float32)
    y2 = y2 * dinv[:, None]
    y2o[...] = jnp.pad(y2, ((0, 0), (0, 64)))


def _out_body(s2, y2, dg0, dg1, b2, o):
    dinv = lax.rsqrt(dg0[...] + dg1[...] + 1.0)
    o[...] = jnp.maximum(
        (s2[..., 0:64] + y2[..., 0:64]) * dinv[:, None] + b2[...][None, :],
        0.0)


def kernel(cat_idx, sub_idx, elem_idx, edge_index,
           emb_cat, emb_sub, emb_elem, W1, b1, W2, b2):
    f32 = jnp.float32
    cat1 = jnp.pad(cat_idx, (0, NP - N))
    sub1 = jnp.pad(sub_idx, (0, NP - N))
    elem1 = jnp.pad(elem_idx, (0, NP - N))
    src2 = jnp.pad(edge_index[0], (0, EP - E),
                   constant_values=NP - 1).reshape(EPB, BE)
    dst2 = jnp.pad(edge_index[1], (0, EP - E),
                   constant_values=NP - 1).reshape(EPB, BE)
    ones_be = jnp.ones((BE,), f32)
    zeros_rt = jnp.zeros((RT,), f32)
    zeros_rc = jnp.zeros((RT, CW), f32)

    x, degp = _embed_deg(cat1, sub1, elem1, dst2, ones_be, zeros_rt,
                         emb_cat, emb_sub, emb_elem)

    dspec0 = pl.BlockSpec((RB,), lambda i: (i,))
    dspec1 = pl.BlockSpec((RB,), lambda i: (GN + i,))
    nat = pl.BlockSpec((RB, 128), lambda i: (i, 0))

    y1 = pl.pallas_call(
        _y1_body,
        grid=(GN,),
        in_specs=[nat, dspec0, dspec1,
                  pl.BlockSpec((96, 128), lambda i: (0, 0))],
        out_specs=nat,
        out_shape=jax.ShapeDtypeStruct((NP, 128), f32),
    )(x, degp, degp, W1)

    s1, _ = _agg4(y1, src2, dst2, zeros_rc)

    y2 = pl.pallas_call(
        _y2_body,
        grid=(GN,),
        in_specs=[nat, nat, dspec0, dspec1,
                  pl.BlockSpec((128,), lambda i: (0,)),
                  pl.BlockSpec((128, 64), lambda i: (0, 0))],
        out_specs=nat,
        out_shape=jax.ShapeDtypeStruct((NP, 128), f32),
    )(s1, y1, degp, degp, b1, W2)

    s2, _ = _agg2(y2, src2, dst2, zeros_rc)

    out = pl.pallas_call(
        _out_body,
        grid=(GN,),
        in_specs=[nat, nat, dspec0, dspec1,
                  pl.BlockSpec((64,), lambda i: (0,))],
        out_specs=pl.BlockSpec((RB, 64), lambda i: (i, 0)),
        out_shape=jax.ShapeDtypeStruct((NP, 64), f32),
    )(s2, y2, degp, degp, b2)

    return out[:N]


# speedup vs baseline: 1.0040x; 1.0040x over previous
import functools

import jax
import jax.numpy as jnp
from jax import lax
from jax.experimental import pallas as pl
from jax.experimental.pallas import tpu as pltpu
from jax.experimental.pallas import tpu_sc as plsc

N = 50000
E = 800000
NP = 51200
EP = 819200
CW = 32
BN = 40
BE = 128
NC, NS = 2, 16
RW = NP // (NC * NS)
RT = NP // NS
EPB = EP // BE
ETR = EP // NS // BE
EHR = EP // (NC * NS) // BE
IB = 40
RB = 2048
GN = NP // RB

_mesh = plsc.VectorSubcoreMesh(core_axis_name="c", subcore_axis_name="s")
_scp = pltpu.CompilerParams(use_tc_tiling_on_sc=False)


@functools.partial(
    pl.kernel,
    out_type=(
        jax.ShapeDtypeStruct((NP, 128), jnp.float32),
        jax.ShapeDtypeStruct((NC * NP,), jnp.float32),
    ),
    mesh=_mesh,
    compiler_params=_scp,
    scratch_types=[
        pltpu.VMEM((RW,), jnp.int32),
        pltpu.VMEM((RW, CW), jnp.float32),
        pltpu.VMEM((EHR, BE), jnp.int32),
        pltpu.VMEM((BE,), jnp.float32),
        pltpu.VMEM_SHARED((NP,), jnp.float32),
        pltpu.SemaphoreType.DMA,
    ],
)
def _embed_deg(cat_h, sub_h, elem_h, dst_h, ones_h, zeros1_h,
               tcat_h, tsub_h, telem_h,
               x_h, degp_h,
               idx_v, rows_v, eidx_v, ones_v, deg_acc, sem):
    cid = lax.axis_index("c")
    tid = lax.axis_index("s")
    wid = tid * NC + cid

    row0 = pl.multiple_of(tid * RT, 8)
    pltpu.sync_copy(zeros1_h, deg_acc.at[pl.ds(row0, RT)])
    pltpu.sync_copy(ones_h, ones_v)
    ebase = pl.multiple_of((cid * NS + tid) * EHR, 8)
    pltpu.sync_copy(dst_h.at[pl.ds(ebase, EHR), :], eidx_v)
    plsc.subcore_barrier()

    def dbody(blk, c):
        descs = []
        for j in range(25):
            descs.append(pltpu.async_copy(
                ones_v, deg_acc.at[eidx_v.at[blk * 25 + j]], sem, add=True))
        for d in descs:
            d.wait()
        return c
    lax.fori_loop(0, EHR // 25, dbody, 0)

    base = pl.multiple_of(wid * RW, 8)
    for t, (tab_h, nidx_h) in enumerate(((tcat_h, cat_h), (tsub_h, sub_h),
                                         (telem_h, elem_h))):
        pltpu.sync_copy(nidx_h.at[pl.ds(base, RW)], idx_v)
        cps = [
            pltpu.async_copy(tab_h.at[idx_v.at[pl.ds(j * BN, BN)]],
                             rows_v.at[pl.ds(j * BN, BN), :], sem)
            for j in range(RW // BN)
        ]
        for cp in cps:
            cp.wait()
        pltpu.sync_copy(rows_v,
                        x_h.at[pl.ds(base, RW), pl.ds(t * CW, CW)])

    plsc.subcore_barrier()
    orow = pl.multiple_of(cid * NP + tid * RT, 8)
    pltpu.sync_copy(deg_acc.at[pl.ds(row0, RT)], degp_h.at[pl.ds(orow, RT)])


def _make_agg(C):
    @functools.partial(
        pl.kernel,
        out_type=(
            jax.ShapeDtypeStruct((NP, 128), jnp.float32),
            jax.ShapeDtypeStruct((C * NP, CW), jnp.float32),
        ),
        mesh=_mesh,
        compiler_params=_scp,
        scratch_types=[
            pltpu.VMEM((IB, BE), jnp.int32),
            pltpu.VMEM((IB, BE), jnp.int32),
            pltpu.VMEM((4, BE, CW), jnp.float32),
            pltpu.VMEM_SHARED((NP, CW), jnp.float32),
            pltpu.SemaphoreType.DMA,
        ],
    )
    def agg(y_h, srcadj_h, dst_h, zeros_h, s_h, ychk_h,
            src_v, dst_v, rows_v, acc, sem):
        cid = lax.axis_index("c")
        tid = lax.axis_index("s")
        cpc = C // 2
        row0 = pl.multiple_of(tid * RT, 8)
        tb = pl.multiple_of(tid * ETR, 8)

        for k in range(cpc):
            chunk = cid * cpc + k
            crow = pl.multiple_of(chunk * NP + tid * RT, 8)
            pltpu.sync_copy(
                y_h.at[pl.ds(row0, RT), pl.ds(chunk * CW, CW)],
                ychk_h.at[pl.ds(crow, RT), :])
        plsc.subcore_barrier()

        for k in range(cpc):
            chunk = cid * cpc + k
            pltpu.sync_copy(zeros_h, acc.at[pl.ds(row0, RT), :])
            plsc.subcore_barrier()

            srow = pl.multiple_of(chunk * EPB + tb, 8)

            def blk_body(blk, c):
                boff = blk * IB
                pltpu.sync_copy(srcadj_h.at[pl.ds(srow + boff, IB), :], src_v)
                pltpu.sync_copy(dst_h.at[pl.ds(tb + boff, IB), :], dst_v)
                descs = {}
                for j in range(3):
                    descs[j] = pltpu.async_copy(
                        ychk_h.at[src_v.at[j]], rows_v.at[j % 4], sem)
                for j in range(IB):
                    descs[j].wait()
                    if j + 3 < IB:
                        descs[j + 3] = pltpu.async_copy(
                            ychk_h.at[src_v.at[j + 3]], rows_v.at[(j + 3) % 4],
                            sem)
                    pltpu.sync_copy(rows_v.at[j % 4], acc.at[dst_v.at[j]],
                                    add=True)
                return c
            lax.fori_loop(0, ETR // IB, blk_body, 0)
            plsc.subcore_barrier()

            pltpu.sync_copy(
                acc.at[pl.ds(row0, RT), :],
                s_h.at[pl.ds(row0, RT), pl.ds(chunk * CW, CW)])
    return agg


_agg4 = _make_agg(4)
_agg2 = _make_agg(2)


def _y1_body(x, dg0, dg1, w1, y1o):
    dinv = lax.rsqrt(dg0[...] + dg1[...] + 1.0)
    xv = x[...]
    xw = (jnp.dot(xv[:, 0:CW], w1[0:CW, :],
                  preferred_element_type=jnp.float32)
          + jnp.dot(xv[:, CW:2 * CW], w1[CW:2 * CW, :],
                    preferred_element_type=jnp.float32)
          + jnp.dot(xv[:, 2 * CW:3 * CW], w1[2 * CW:3 * CW, :],
                    preferred_element_type=jnp.float32))
    y1o[...] = xw * dinv[:, None]


def _y2_body(s1, y1, dg0, dg1, b1, w2, y2o):
    dinv = lax.rsqrt(dg0[...] + dg1[...] + 1.0)
    t = jnp.maximum((s1[...] + y1[...]) * dinv[:, None] + b1[...][None, :],
                    0.0)
    y2 = jnp.dot(t, w2[...], preferred_element_type=jnp.float32)
    y2 = y2 * dinv[:, None]
    y2o[...] = jnp.pad(y2, ((0, 0), (0, 64)))


def _out_body(s2, y2, dg0, dg1, b2, o):
    dinv = lax.rsqrt(dg0[...] + dg1[...] + 1.0)
    o[...] = jnp.maximum(
        (s2[..., 0:64] + y2[..., 0:64]) * dinv[:, None] + b2[...][None, :],
        0.0)


def kernel(cat_idx, sub_idx, elem_idx, edge_index,
           emb_cat, emb_sub, emb_elem, W1, b1, W2, b2):
    f32 = jnp.float32
    cat1 = jnp.pad(cat_idx, (0, NP - N))
    sub1 = jnp.pad(sub_idx, (0, NP - N))
    elem1 = jnp.pad(elem_idx, (0, NP - N))
    src_p = jnp.pad(edge_index[0], (0, EP - E), constant_values=NP - 1)
    srcadj4 = (src_p[None, :]
               + (jnp.arange(4, dtype=jnp.int32) * NP)[:, None]
               ).reshape(4 * EPB, BE)
    srcadj2 = srcadj4[:2 * EPB]
    dst2 = jnp.pad(edge_index[1], (0, EP - E),
                   constant_values=NP - 1).reshape(EPB, BE)
    ones_be = jnp.ones((BE,), f32)
    zeros_rt = jnp.zeros((RT,), f32)
    zeros_rc = jnp.zeros((RT, CW), f32)

    x, degp = _embed_deg(cat1, sub1, elem1, dst2, ones_be, zeros_rt,
                         emb_cat, emb_sub, emb_elem)

    dspec0 = pl.BlockSpec((RB,), lambda i: (i,))
    dspec1 = pl.BlockSpec((RB,), lambda i: (GN + i,))
    nat = pl.BlockSpec((RB, 128), lambda i: (i, 0))

    y1 = pl.pallas_call(
        _y1_body,
        grid=(GN,),
        in_specs=[nat, dspec0, dspec1,
                  pl.BlockSpec((96, 128), lambda i: (0, 0))],
        out_specs=nat,
        out_shape=jax.ShapeDtypeStruct((NP, 128), f32),
    )(x, degp, degp, W1)

    s1, _ = _agg4(y1, srcadj4, dst2, zeros_rc)

    y2 = pl.pallas_call(
        _y2_body,
        grid=(GN,),
        in_specs=[nat, nat, dspec0, dspec1,
                  pl.BlockSpec((128,), lambda i: (0,)),
                  pl.BlockSpec((128, 64), lambda i: (0, 0))],
        out_specs=nat,
        out_shape=jax.ShapeDtypeStruct((NP, 128), f32),
    )(s1, y1, degp, degp, b1, W2)

    s2, _ = _agg2(y2, srcadj2, dst2, zeros_rc)

    out = pl.pallas_call(
        _out_body,
        grid=(GN,),
        in_specs=[nat, nat, dspec0, dspec1,
                  pl.BlockSpec((64,), lambda i: (0,))],
        out_specs=pl.BlockSpec((RB, 64), lambda i: (i, 0)),
        out_shape=jax.ShapeDtypeStruct((NP, 64), f32),
    )(s2, y2, degp, degp, b2)

    return out[:N]


# speedup vs baseline: 1.3852x; 1.3797x over previous
import functools

import jax
import jax.numpy as jnp
from jax import lax
from jax.experimental import pallas as pl
from jax.experimental.pallas import tpu as pltpu
from jax.experimental.pallas import tpu_sc as plsc

N = 50000
E = 800000
NP = 51200
EP = 819200
CW = 32
BN = 40
BE = 128
NC, NS = 2, 16
RW = NP // (NC * NS)
RT = NP // NS
EPB = EP // BE
ETR = EP // NS // BE
EHR = EP // (NC * NS) // BE
IB = 40
RB = 2048
GN = NP // RB

_mesh = plsc.VectorSubcoreMesh(core_axis_name="c", subcore_axis_name="s")
_scp = pltpu.CompilerParams(use_tc_tiling_on_sc=False)


@functools.partial(
    pl.kernel,
    out_type=(
        jax.ShapeDtypeStruct((NP, 128), jnp.float32),
        jax.ShapeDtypeStruct((NC * NP,), jnp.float32),
    ),
    mesh=_mesh,
    compiler_params=_scp,
    scratch_types=[
        pltpu.VMEM((RW,), jnp.int32),
        pltpu.VMEM((RW, CW), jnp.float32),
        pltpu.VMEM((EHR, BE), jnp.int32),
        pltpu.VMEM((BE,), jnp.float32),
        pltpu.VMEM_SHARED((NP,), jnp.float32),
        pltpu.SemaphoreType.DMA,
    ],
)
def _embed_deg(cat_h, sub_h, elem_h, dst_h, ones_h, zeros1_h,
               tcat_h, tsub_h, telem_h,
               x_h, degp_h,
               idx_v, rows_v, eidx_v, ones_v, deg_acc, sem):
    cid = lax.axis_index("c")
    tid = lax.axis_index("s")
    wid = tid * NC + cid

    row0 = pl.multiple_of(tid * RT, 8)
    pltpu.sync_copy(zeros1_h, deg_acc.at[pl.ds(row0, RT)])
    pltpu.sync_copy(ones_h, ones_v)
    ebase = pl.multiple_of((cid * NS + tid) * EHR, 8)
    pltpu.sync_copy(dst_h.at[pl.ds(ebase, EHR), :], eidx_v)
    plsc.subcore_barrier()

    def dbody(blk, c):
        descs = []
        for j in range(25):
            descs.append(pltpu.async_copy(
                ones_v, deg_acc.at[eidx_v.at[blk * 25 + j]], sem, add=True))
        for d in descs:
            d.wait()
        return c
    lax.fori_loop(0, EHR // 25, dbody, 0)

    base = pl.multiple_of(wid * RW, 8)
    for t, (tab_h, nidx_h) in enumerate(((tcat_h, cat_h), (tsub_h, sub_h),
                                         (telem_h, elem_h))):
        pltpu.sync_copy(nidx_h.at[pl.ds(base, RW)], idx_v)
        cps = [
            pltpu.async_copy(tab_h.at[idx_v.at[pl.ds(j * BN, BN)]],
                             rows_v.at[pl.ds(j * BN, BN), :], sem)
            for j in range(RW // BN)
        ]
        for cp in cps:
            cp.wait()
        pltpu.sync_copy(rows_v,
                        x_h.at[pl.ds(base, RW), pl.ds(t * CW, CW)])

    plsc.subcore_barrier()
    orow = pl.multiple_of(cid * NP + tid * RT, 8)
    pltpu.sync_copy(deg_acc.at[pl.ds(row0, RT)], degp_h.at[pl.ds(orow, RT)])


def _make_agg(C):
    @functools.partial(
        pl.kernel,
        out_type=jax.ShapeDtypeStruct((C * NP, CW), jnp.float32),
        mesh=_mesh,
        compiler_params=_scp,
        scratch_types=[
            pltpu.VMEM((IB, BE), jnp.int32),
            pltpu.VMEM((IB, BE), jnp.int32),
            pltpu.VMEM((4, BE, CW), jnp.float32),
            pltpu.VMEM_SHARED((NP, CW), jnp.float32),
            pltpu.SemaphoreType.DMA,
            pltpu.SemaphoreType.DMA,
        ],
    )
    def agg(y_h, srcadj_h, dst_h, zeros_h, s_h,
            src_v, dst_v, rows_v, acc, gsem, ssem):
        cid = lax.axis_index("c")
        tid = lax.axis_index("s")
        cpc = C // 2
        row0 = pl.multiple_of(tid * RT, 8)
        tb = pl.multiple_of(tid * ETR, 8)

        for k in range(cpc):
            chunk = cid * cpc + k
            pltpu.sync_copy(zeros_h, acc.at[pl.ds(row0, RT), :])
            plsc.subcore_barrier()

            srow = pl.multiple_of(chunk * EPB + tb, 8)

            def blk_body(blk, c):
                boff = blk * IB
                pltpu.sync_copy(srcadj_h.at[pl.ds(srow + boff, IB), :], src_v)
                pltpu.sync_copy(dst_h.at[pl.ds(tb + boff, IB), :], dst_v)
                gd, sd = {}, {}
                for j in range(3):
                    gd[j] = pltpu.async_copy(
                        y_h.at[src_v.at[j]], rows_v.at[j % 4], gsem)
                for j in range(IB):
                    gd[j].wait()
                    sd[j] = pltpu.async_copy(
                        rows_v.at[j % 4], acc.at[dst_v.at[j]], ssem, add=True)
                    if j + 3 < IB:
                        if j >= 1:
                            sd[j - 1].wait()
                        gd[j + 3] = pltpu.async_copy(
                            y_h.at[src_v.at[j + 3]], rows_v.at[(j + 3) % 4],
                            gsem)
                for j in range(IB - 4, IB):
                    sd[j].wait()
                return c
            lax.fori_loop(0, ETR // IB, blk_body, 0)
            plsc.subcore_barrier()

            orow = pl.multiple_of(chunk * NP + tid * RT, 8)
            pltpu.sync_copy(acc.at[pl.ds(row0, RT), :],
                            s_h.at[pl.ds(orow, RT), :])
    return agg


_agg4 = _make_agg(4)
_agg2 = _make_agg(2)


def _y1_body(x, dg0, dg1, w1r, y1o):
    dinv = lax.rsqrt(dg0[...] + dg1[...] + 1.0)
    xv = x[...]
    w = w1r[0]
    xw = (jnp.dot(xv[:, 0:CW], w[0:CW, :],
                  preferred_element_type=jnp.float32)
          + jnp.dot(xv[:, CW:2 * CW], w[CW:2 * CW, :],
                    preferred_element_type=jnp.float32)
          + jnp.dot(xv[:, 2 * CW:3 * CW], w[2 * CW:3 * CW, :],
                    preferred_element_type=jnp.float32))
    y1o[...] = xw * dinv[:, None]


def _y2_body(s10, s11, s12, s13, y10, y11, y12, y13, dg0, dg1, b1, w2r, y2o):
    dinv = lax.rsqrt(dg0[...] + dg1[...] + 1.0)
    w = w2r[0]
    acc = jnp.zeros((RB, CW), jnp.float32)
    for cc, (sc, yc) in enumerate(((s10, y10), (s11, y11),
                                   (s12, y12), (s13, y13))):
        t = jnp.maximum((sc[...] + yc[...]) * dinv[:, None]
                        + b1[...][cc * CW:(cc + 1) * CW][None, :], 0.0)
        acc = acc + jnp.dot(t, w[cc * CW:(cc + 1) * CW, :],
                            preferred_element_type=jnp.float32)
    y2o[...] = acc * dinv[:, None]


def _out_body(s20, s21, y20, y21, dg0, dg1, b2, o):
    dinv = lax.rsqrt(dg0[...] + dg1[...] + 1.0)
    o0 = jnp.maximum((s20[...] + y20[...]) * dinv[:, None]
                     + b2[...][0:CW][None, :], 0.0)
    o1 = jnp.maximum((s21[...] + y21[...]) * dinv[:, None]
                     + b2[...][CW:2 * CW][None, :], 0.0)
    o[...] = jnp.concatenate([o0, o1], axis=1)


def kernel(cat_idx, sub_idx, elem_idx, edge_index,
           emb_cat, emb_sub, emb_elem, W1, b1, W2, b2):
    f32 = jnp.float32
    cat1 = jnp.pad(cat_idx, (0, NP - N))
    sub1 = jnp.pad(sub_idx, (0, NP - N))
    elem1 = jnp.pad(elem_idx, (0, NP - N))
    src_p = jnp.pad(edge_index[0], (0, EP - E), constant_values=NP - 1)
    srcadj4 = (src_p[None, :]
               + (jnp.arange(4, dtype=jnp.int32) * NP)[:, None]
               ).reshape(4 * EPB, BE)
    srcadj2 = srcadj4[:2 * EPB]
    dst2 = jnp.pad(edge_index[1], (0, EP - E),
                   constant_values=NP - 1).reshape(EPB, BE)
    ones_be = jnp.ones((BE,), f32)
    zeros_rt = jnp.zeros((RT,), f32)
    zeros_rc = jnp.zeros((RT, CW), f32)
    w1r = jnp.swapaxes(W1.reshape(96, 4, CW), 0, 1)
    w2r = jnp.swapaxes(W2.reshape(128, 2, CW), 0, 1)

    x, degp = _embed_deg(cat1, sub1, elem1, dst2, ones_be, zeros_rt,
                         emb_cat, emb_sub, emb_elem)

    dspec0 = pl.BlockSpec((RB,), lambda i, c: (i,))
    dspec1 = pl.BlockSpec((RB,), lambda i, c: (GN + i,))

    def cspec(cc):
        return pl.BlockSpec((RB, CW), lambda i, c, cc=cc: (cc * GN + i, 0))

    y1 = pl.pallas_call(
        _y1_body,
        grid=(GN, 4),
        in_specs=[pl.BlockSpec((RB, 128), lambda i, c: (i, 0)),
                  dspec0, dspec1,
                  pl.BlockSpec((1, 96, CW), lambda i, c: (c, 0, 0))],
        out_specs=pl.BlockSpec((RB, CW), lambda i, c: (c * GN + i, 0)),
        out_shape=jax.ShapeDtypeStruct((4 * NP, CW), f32),
    )(x, degp, degp, w1r)

    s1 = _agg4(y1, srcadj4, dst2, zeros_rc)

    y2 = pl.pallas_call(
        _y2_body,
        grid=(GN, 2),
        in_specs=([cspec(cc) for cc in range(4)]
                  + [cspec(cc) for cc in range(4)]
                  + [dspec0, dspec1,
                     pl.BlockSpec((128,), lambda i, c: (0,)),
                     pl.BlockSpec((1, 128, CW), lambda i, c: (c, 0, 0))]),
        out_specs=pl.BlockSpec((RB, CW), lambda i, c: (c * GN + i, 0)),
        out_shape=jax.ShapeDtypeStruct((2 * NP, CW), f32),
    )(s1, s1, s1, s1, y1, y1, y1, y1, degp, degp, b1, w2r)

    s2 = _agg2(y2, srcadj2, dst2, zeros_rc)

    def cspec1(cc):
        return pl.BlockSpec((RB, CW), lambda i, cc=cc: (cc * GN + i, 0))

    out = pl.pallas_call(
        _out_body,
        grid=(GN,),
        in_specs=([cspec1(0), cspec1(1), cspec1(0), cspec1(1)]
                  + [pl.BlockSpec((RB,), lambda i: (i,)),
                     pl.BlockSpec((RB,), lambda i: (GN + i,)),
                     pl.BlockSpec((64,), lambda i: (0,))]),
        out_specs=pl.BlockSpec((RB, 64), lambda i: (i, 0)),
        out_shape=jax.ShapeDtypeStruct((NP, 64), f32),
    )(s2, s2, y2, y2, degp, degp, b2)

    return out[:N]


# speedup vs baseline: 1.4518x; 1.0481x over previous
import functools

import jax
import jax.numpy as jnp
from jax import lax
from jax.experimental import pallas as pl
from jax.experimental.pallas import tpu as pltpu
from jax.experimental.pallas import tpu_sc as plsc

N = 50000
E = 800000
NP = 51200
EP = 819200
CW = 32
BN = 40
BE = 128
NC, NS = 2, 16
RW = NP // (NC * NS)
RT = NP // NS
EPB = EP // BE
ETR = EP // NS // BE
EHR = EP // (NC * NS) // BE
IB = 40
RB = 2048
GN = NP // RB

_mesh = plsc.VectorSubcoreMesh(core_axis_name="c", subcore_axis_name="s")
_scp = pltpu.CompilerParams(use_tc_tiling_on_sc=False)


@functools.partial(
    pl.kernel,
    out_type=(
        jax.ShapeDtypeStruct((NP, 128), jnp.float32),
        jax.ShapeDtypeStruct((NC * NP,), jnp.float32),
    ),
    mesh=_mesh,
    compiler_params=_scp,
    scratch_types=[
        pltpu.VMEM((RW,), jnp.int32),
        pltpu.VMEM((RW, CW), jnp.float32),
        pltpu.VMEM((EHR, BE), jnp.int32),
        pltpu.VMEM((BE,), jnp.float32),
        pltpu.VMEM_SHARED((NP,), jnp.float32),
        pltpu.SemaphoreType.DMA,
    ],
)
def _embed_deg(cat_h, sub_h, elem_h, dst_h, ones_h, zeros1_h,
               tcat_h, tsub_h, telem_h,
               x_h, degp_h,
               idx_v, rows_v, eidx_v, ones_v, deg_acc, sem):
    cid = lax.axis_index("c")
    tid = lax.axis_index("s")
    wid = tid * NC + cid

    row0 = pl.multiple_of(tid * RT, 8)
    pltpu.sync_copy(zeros1_h, deg_acc.at[pl.ds(row0, RT)])
    pltpu.sync_copy(ones_h, ones_v)
    ebase = pl.multiple_of((cid * NS + tid) * EHR, 8)
    pltpu.sync_copy(dst_h.at[pl.ds(ebase, EHR), :], eidx_v)
    plsc.subcore_barrier()

    def dbody(blk, c):
        descs = []
        for j in range(25):
            descs.append(pltpu.async_copy(
                ones_v, deg_acc.at[eidx_v.at[blk * 25 + j]], sem, add=True))
        for d in descs:
            d.wait()
        return c
    lax.fori_loop(0, EHR // 25, dbody, 0)

    base = pl.multiple_of(wid * RW, 8)
    for t, (tab_h, nidx_h) in enumerate(((tcat_h, cat_h), (tsub_h, sub_h),
                                         (telem_h, elem_h))):
        pltpu.sync_copy(nidx_h.at[pl.ds(base, RW)], idx_v)
        cps = [
            pltpu.async_copy(tab_h.at[idx_v.at[pl.ds(j * BN, BN)]],
                             rows_v.at[pl.ds(j * BN, BN), :], sem)
            for j in range(RW // BN)
        ]
        for cp in cps:
            cp.wait()
        pltpu.sync_copy(rows_v,
                        x_h.at[pl.ds(base, RW), pl.ds(t * CW, CW)])

    plsc.subcore_barrier()
    orow = pl.multiple_of(cid * NP + tid * RT, 8)
    pltpu.sync_copy(deg_acc.at[pl.ds(row0, RT)], degp_h.at[pl.ds(orow, RT)])


def _make_agg(C):
    @functools.partial(
        pl.kernel,
        out_type=jax.ShapeDtypeStruct((C * NP, CW), jnp.float32),
        mesh=_mesh,
        compiler_params=_scp,
        scratch_types=[
            pltpu.VMEM((IB, BE), jnp.int32),
            pltpu.VMEM((IB, BE), jnp.int32),
            pltpu.VMEM((4, BE, CW), jnp.float32),
            pltpu.VMEM_SHARED((NP, CW), jnp.float32),
            pltpu.SemaphoreType.DMA,
        ],
    )
    def agg(y_h, srcadj_h, dst_h, zeros_h, s_h,
            src_v, dst_v, rows_v, acc, gsem):
        cid = lax.axis_index("c")
        tid = lax.axis_index("s")
        cpc = C // 2
        row0 = pl.multiple_of(tid * RT, 8)
        tb = pl.multiple_of(tid * ETR, 8)

        for k in range(cpc):
            chunk = cid * cpc + k
            pltpu.sync_copy(zeros_h, acc.at[pl.ds(row0, RT), :])
            plsc.subcore_barrier()

            srow = pl.multiple_of(chunk * EPB + tb, 8)

            def blk_body(blk, c):
                boff = blk * IB
                pltpu.sync_copy(srcadj_h.at[pl.ds(srow + boff, IB), :], src_v)
                pltpu.sync_copy(dst_h.at[pl.ds(tb + boff, IB), :], dst_v)
                gd = {}
                for j in range(3):
                    gd[j] = pltpu.async_copy(
                        y_h.at[src_v.at[j]], rows_v.at[j % 4], gsem)
                for j in range(IB):
                    gd[j].wait()
                    if j + 3 < IB:
                        gd[j + 3] = pltpu.async_copy(
                            y_h.at[src_v.at[j + 3]], rows_v.at[(j + 3) % 4],
                            gsem)
                    pltpu.sync_copy(rows_v.at[j % 4], acc.at[dst_v.at[j]],
                                    add=True)
                return c
            lax.fori_loop(0, ETR // IB, blk_body, 0)
            plsc.subcore_barrier()

            orow = pl.multiple_of(chunk * NP + tid * RT, 8)
            pltpu.sync_copy(acc.at[pl.ds(row0, RT), :],
                            s_h.at[pl.ds(orow, RT), :])
    return agg


_agg4 = _make_agg(4)
_agg2 = _make_agg(2)


def _y1_body(x, dg0, dg1, w1, y1o):
    dinv = lax.rsqrt(dg0[...] + dg1[...] + 1.0)
    xv = x[...]
    xw = (jnp.dot(xv[:, 0:CW], w1[0:CW, :],
                  preferred_element_type=jnp.float32)
          + jnp.dot(xv[:, CW:2 * CW], w1[CW:2 * CW, :],
                    preferred_element_type=jnp.float32)
          + jnp.dot(xv[:, 2 * CW:3 * CW], w1[2 * CW:3 * CW, :],
                    preferred_element_type=jnp.float32))
    y1o[...] = xw * dinv[:, None]


def _y2_body(s1, y1, dg0, dg1, b1, w2, y2o):
    dinv = lax.rsqrt(dg0[...] + dg1[...] + 1.0)
    t = jnp.maximum((s1[...] + y1[...]) * dinv[:, None] + b1[...][None, :],
                    0.0)
    y2 = jnp.dot(t, w2[...], preferred_element_type=jnp.float32)
    y2o[...] = y2 * dinv[:, None]


def _out_body(s2, y2, dg0, dg1, b2, o):
    dinv = lax.rsqrt(dg0[...] + dg1[...] + 1.0)
    o[...] = jnp.maximum((s2[...] + y2[...]) * dinv[:, None]
                         + b2[...][None, :], 0.0)


def kernel(cat_idx, sub_idx, elem_idx, edge_index,
           emb_cat, emb_sub, emb_elem, W1, b1, W2, b2):
    f32 = jnp.float32
    cat1 = jnp.pad(cat_idx, (0, NP - N))
    sub1 = jnp.pad(sub_idx, (0, NP - N))
    elem1 = jnp.pad(elem_idx, (0, NP - N))
    src_p = jnp.pad(edge_index[0], (0, EP - E), constant_values=NP - 1)
    srcadj4 = (src_p[None, :]
               + (jnp.arange(4, dtype=jnp.int32) * NP)[:, None]
               ).reshape(4 * EPB, BE)
    srcadj2 = srcadj4[:2 * EPB]
    dst2 = jnp.pad(edge_index[1], (0, EP - E),
                   constant_values=NP - 1).reshape(EPB, BE)
    ones_be = jnp.ones((BE,), f32)
    zeros_rt = jnp.zeros((RT,), f32)
    zeros_rc = jnp.zeros((RT, CW), f32)
    x, degp = _embed_deg(cat1, sub1, elem1, dst2, ones_be, zeros_rt,
                         emb_cat, emb_sub, emb_elem)

    dspec0 = pl.BlockSpec((RB,), lambda i: (i,))
    dspec1 = pl.BlockSpec((RB,), lambda i: (GN + i,))
    nat = pl.BlockSpec((RB, 128), lambda i: (i, 0))

    def to_chunks(a, c):
        return jnp.transpose(a.reshape(NP, c, CW), (1, 0, 2)).reshape(
            c * NP, CW)

    def to_natural(a, c):
        return jnp.transpose(a.reshape(c, NP, CW), (1, 0, 2)).reshape(
            NP, c * CW)

    y1n = pl.pallas_call(
        _y1_body,
        grid=(GN,),
        in_specs=[nat, dspec0, dspec1,
                  pl.BlockSpec((96, 128), lambda i: (0, 0))],
        out_specs=nat,
        out_shape=jax.ShapeDtypeStruct((NP, 128), f32),
    )(x, degp, degp, W1)

    s1 = _agg4(to_chunks(y1n, 4), srcadj4, dst2, zeros_rc)

    y2n = pl.pallas_call(
        _y2_body,
        grid=(GN,),
        in_specs=[nat, nat, dspec0, dspec1,
                  pl.BlockSpec((128,), lambda i: (0,)),
                  pl.BlockSpec((128, 64), lambda i: (0, 0))],
        out_specs=pl.BlockSpec((RB, 64), lambda i: (i, 0)),
        out_shape=jax.ShapeDtypeStruct((NP, 64), f32),
    )(to_natural(s1, 4), y1n, degp, degp, b1, W2)

    s2 = _agg2(to_chunks(y2n, 2), srcadj2, dst2, zeros_rc)

    nat64 = pl.BlockSpec((RB, 64), lambda i: (i, 0))
    out = pl.pallas_call(
        _out_body,
        grid=(GN,),
        in_specs=[nat64, nat64, dspec0, dspec1,
                  pl.BlockSpec((64,), lambda i: (0,))],
        out_specs=nat64,
        out_shape=jax.ShapeDtypeStruct((NP, 64), f32),
    )(to_natural(s2, 2), y2n, degp, degp, b2)

    return out[:N]
